# Initial kernel scaffold; baseline (speedup 1.0000x reference)
import jax
import jax.numpy as jnp
from jax.experimental import pallas as pl


def kernel(embd, trgt, mask):
    raise NotImplementedError("write your pallas kernel here")


# speedup vs baseline: 5.1185x; 5.1185x over previous
import functools

import jax
import jax.numpy as jnp
from jax import lax
from jax.experimental import pallas as pl
from jax.experimental.pallas import tpu as pltpu
from jax.experimental.pallas import tpu_sc as plsc

K = 64
C = 16
N = 32 * 256 * 256
ALPHA = 1.0
BETA = 1.0
GAMMA = 0.001
DELTA_D = 1.5

NC = 2
NS = 16
L = 16
NW = NC * NS
KL = K * L
VPW = N // NW
BLK = 4096

_MESH = plsc.VectorSubcoreMesh(
    core_axis_name="c", subcore_axis_name="s", num_cores=NC, num_subcores=NS
)


def _worker_id():
    return lax.axis_index("s") * NC + lax.axis_index("c")


@functools.partial(
    pl.kernel,
    out_type=[
        jax.ShapeDtypeStruct((NW, C * KL), jnp.float32),
        jax.ShapeDtypeStruct((NW, KL), jnp.float32),
    ],
    mesh=_MESH,
    compiler_params=pltpu.CompilerParams(needs_layout_passes=False),
    scratch_types=[
        pltpu.VMEM((C, BLK), jnp.float32),
        pltpu.VMEM((BLK,), jnp.int32),
        pltpu.VMEM((BLK,), jnp.int32),
        pltpu.VMEM((C * KL,), jnp.float32),
        pltpu.VMEM((KL,), jnp.float32),
        pltpu.SemaphoreType.DMA,
    ],
)
def _sc_pass1(e_hbm, t_hbm, m_hbm, sums_out, cnt_out,
              xbuf, tbuf, mbuf, acc, cacc, sem):
    wid = _worker_id()
    base = wid * VPW
    lane = lax.broadcasted_iota(jnp.int32, (L,), 0)
    zero = jnp.zeros((L,), jnp.float32)
    ones = jnp.ones((L,), jnp.float32)

    def zero_body(i, _):
        sl = pl.ds(i * L, L)
        cacc[sl] = zero
        for c in range(C):
            acc[pl.ds(c * KL + i * L, L)] = zero
        return 0

    lax.fori_loop(0, KL // L, zero_body, 0)

    def chunk_body(j, _):
        off = base + j * BLK
        cps = [pltpu.async_copy(e_hbm.at[c, pl.ds(off, BLK)], xbuf.at[c], sem)
               for c in range(C)]
        cps.append(pltpu.async_copy(t_hbm.at[pl.ds(off, BLK)], tbuf, sem))
        cps.append(pltpu.async_copy(m_hbm.at[pl.ds(off, BLK)], mbuf, sem))
        for cp in cps:
            cp.wait()

        def vec_body(i, _):
            sl = pl.ds(i * L, L)
            seg = jnp.where(mbuf[sl] > 0, tbuf[sl], 0)
            addr = seg * L + lane
            plsc.addupdate_scatter(cacc, [addr], ones)
            for c in range(C):
                plsc.addupdate_scatter(acc, [addr + (c * KL)], xbuf[c, sl])
            return 0

        lax.fori_loop(0, BLK // L, vec_body, 0)
        return 0

    lax.fori_loop(0, VPW // BLK, chunk_body, 0)
    pltpu.sync_copy(acc, sums_out.at[wid])
    pltpu.sync_copy(cacc, cnt_out.at[wid])


@functools.partial(
    pl.kernel,
    out_type=jax.ShapeDtypeStruct((NW, KL), jnp.float32),
    mesh=_MESH,
    compiler_params=pltpu.CompilerParams(needs_layout_passes=False),
    scratch_types=[
        pltpu.VMEM((C, BLK), jnp.float32),
        pltpu.VMEM((BLK,), jnp.int32),
        pltpu.VMEM((BLK,), jnp.int32),
        pltpu.VMEM((C * K,), jnp.float32),
        pltpu.VMEM((KL,), jnp.float32),
        pltpu.SemaphoreType.DMA,
    ],
)
def _sc_pass2(e_hbm, t_hbm, m_hbm, means_hbm, term_out,
              xbuf, tbuf, mbuf, mtab, tacc, sem):
    wid = _worker_id()
    base = wid * VPW
    lane = lax.broadcasted_iota(jnp.int32, (L,), 0)
    zero = jnp.zeros((L,), jnp.float32)

    pltpu.sync_copy(means_hbm, mtab)

    def zero_body(i, _):
        tacc[pl.ds(i * L, L)] = zero
        return 0

    lax.fori_loop(0, KL // L, zero_body, 0)

    def chunk_body(j, _):
        off = base + j * BLK
        cps = [pltpu.async_copy(e_hbm.at[c, pl.ds(off, BLK)], xbuf.at[c], sem)
               for c in range(C)]
        cps.append(pltpu.async_copy(t_hbm.at[pl.ds(off, BLK)], tbuf, sem))
        cps.append(pltpu.async_copy(m_hbm.at[pl.ds(off, BLK)], mbuf, sem))
        for cp in cps:
            cp.wait()

        def vec_body(i, _):
            sl = pl.ds(i * L, L)
            seg = jnp.where(mbuf[sl] > 0, tbuf[sl], 0)
            d = zero
            for c in range(C):
                mv = plsc.load_gather(mtab, [seg + (c * K)])
                d = d + jnp.abs(xbuf[c, sl] - mv)
            plsc.addupdate_scatter(tacc, [seg * L + lane], d * d)
            return 0

        lax.fori_loop(0, BLK // L, vec_body, 0)
        return 0

    lax.fori_loop(0, VPW // BLK, chunk_body, 0)
    pltpu.sync_copy(tacc, term_out.at[wid])


def _fold_matrices():
    jv = lax.broadcasted_iota(jnp.int32, (KL, K), 0)
    kv = lax.broadcasted_iota(jnp.int32, (KL, K), 1)
    F = ((jv // L) == kv).astype(jnp.float32)
    kv2 = lax.broadcasted_iota(jnp.int32, (K, KL), 0)
    jv2 = lax.broadcasted_iota(jnp.int32, (K, KL), 1)
    F2 = ((jv2 // L) == kv2).astype(jnp.float32)
    return F, F2


def _combine1_body(sums_ref, cnt_ref, means_ref, scal_ref):
    ones_w = jnp.ones((1, NW), jnp.float32)
    red = jnp.dot(ones_w, sums_ref[...], preferred_element_type=jnp.float32)
    cred = jnp.dot(ones_w, cnt_ref[...], preferred_element_type=jnp.float32)
    F, F2 = _fold_matrices()
    nt = (((1,), (1,)), ((), ()))
    cnt_row = jnp.dot(cred, F, preferred_element_type=jnp.float32)
    cnt_col = lax.dot_general(F2, cred, nt,
                              preferred_element_type=jnp.float32)
    safe_row = jnp.maximum(cnt_row, 1.0)
    safe_col = jnp.maximum(cnt_col, 1.0)
    obj_row = (cnt_row > 0.0) & (lax.broadcasted_iota(jnp.int32, (1, K), 1) > 0)
    obj_col = (cnt_col > 0.0) & (lax.broadcasted_iota(jnp.int32, (K, 1), 0) > 0)
    n_obj = jnp.sum(obj_row.astype(jnp.float32))
    dist = jnp.zeros((K, K), jnp.float32)
    nrm = jnp.zeros((K, 1), jnp.float32)
    for c in range(C):
        seg_c = red[:, c * KL:(c + 1) * KL]
        row_c = jnp.dot(seg_c, F, preferred_element_type=jnp.float32) / safe_row
        col_c = lax.dot_general(F2, seg_c, nt,
                                preferred_element_type=jnp.float32) / safe_col
        means_ref[c:c + 1, :] = row_c
        dist = dist + jnp.abs(col_c - row_c)
        nrm = nrm + jnp.abs(col_c)
    margin = jnp.maximum(2.0 * DELTA_D - dist, 0.0)
    ii = lax.broadcasted_iota(jnp.int32, (K, K), 0)
    jj = lax.broadcasted_iota(jnp.int32, (K, K), 1)
    pair = obj_col & obj_row & (ii != jj)
    loss_ext = jnp.sum(jnp.where(pair, margin * margin, 0.0))
    loss_ext = loss_ext / jnp.maximum(1.0, n_obj * (n_obj - 1.0))
    loss_nrm = jnp.sum(jnp.where(obj_col, nrm, 0.0)) / jnp.maximum(1.0, n_obj)
    scal_ref[...] = jnp.reshape(BETA * loss_ext + GAMMA * loss_nrm, (1, 1))


def _combine2_body(term_ref, cnt_ref, scal_ref, out_ref):
    ones_w = jnp.ones((1, NW), jnp.float32)
    tred = jnp.dot(ones_w, term_ref[...], preferred_element_type=jnp.float32)
    cred = jnp.dot(ones_w, cnt_ref[...], preferred_element_type=jnp.float32)
    F, _ = _fold_matrices()
    t_row = jnp.dot(tred, F, preferred_element_type=jnp.float32)
    cnt_row = jnp.dot(cred, F, preferred_element_type=jnp.float32)
    per_obj = t_row / jnp.maximum(cnt_row, 1.0)
    obj_row = (cnt_row > 0.0) & (lax.broadcasted_iota(jnp.int32, (1, K), 1) > 0)
    n_obj = jnp.sum(obj_row.astype(jnp.float32))
    loss_int = jnp.sum(jnp.where(obj_row, per_obj, 0.0))
    loss_int = loss_int / jnp.maximum(1.0, n_obj)
    out_ref[...] = jnp.reshape(ALPHA * loss_int, (1, 1)) + scal_ref[...]


def _combine1(sums_part, cnt_part):
    return pl.pallas_call(
        _combine1_body,
        out_shape=[
            jax.ShapeDtypeStruct((C, K), jnp.float32),
            jax.ShapeDtypeStruct((1, 1), jnp.float32),
        ],
    )(sums_part, cnt_part)


def _combine2(term_part, cnt_part, scal):
    return pl.pallas_call(
        _combine2_body,
        out_shape=jax.ShapeDtypeStruct((1, 1), jnp.float32),
    )(term_part, cnt_part, scal)


def kernel(embd, trgt, mask):
    e2 = embd.reshape(C, N)
    t1 = trgt.reshape(N).astype(jnp.int32)
    m1 = mask.reshape(N).astype(jnp.int32)
    sums_part, cnt_part = _sc_pass1(e2, t1, m1)
    means_ck, scal = _combine1(sums_part, cnt_part)
    term_part = _sc_pass2(e2, t1, m1, means_ck.reshape(C * K))
    out = _combine2(term_part, cnt_part, scal)
    return out[0, 0]


# speedup vs baseline: 5.1476x; 1.0057x over previous
import functools

import jax
import jax.numpy as jnp
from jax import lax
from jax.experimental import pallas as pl
from jax.experimental.pallas import tpu as pltpu
from jax.experimental.pallas import tpu_sc as plsc

K = 64
C = 16
N = 32 * 256 * 256
ALPHA = 1.0
BETA = 1.0
GAMMA = 0.001
DELTA_D = 1.5

NC = 2
NS = 16
L = 16
NW = NC * NS
KL = K * L
VPW = N // NW
BLK = 2048
NCHUNK = VPW // BLK

_MESH = plsc.VectorSubcoreMesh(
    core_axis_name="c", subcore_axis_name="s", num_cores=NC, num_subcores=NS
)


def _worker_id():
    return lax.axis_index("s") * NC + lax.axis_index("c")


def _issue_chunk(e_hbm, t_hbm, m_hbm, xbuf, tbuf, mbuf, b, off, sem):
    for c in range(C):
        pltpu.async_copy(e_hbm.at[c, pl.ds(off, BLK)], xbuf.at[b, c], sem)
    pltpu.async_copy(t_hbm.at[pl.ds(off, BLK)], tbuf.at[b], sem)
    pltpu.async_copy(m_hbm.at[pl.ds(off, BLK)], mbuf.at[b], sem)


def _drain_chunk(e_hbm, t_hbm, m_hbm, xbuf, tbuf, mbuf, b, sem):
    for c in range(C):
        pltpu.make_async_copy(
            e_hbm.at[c, pl.ds(0, BLK)], xbuf.at[b, c], sem).wait()
    pltpu.make_async_copy(t_hbm.at[pl.ds(0, BLK)], tbuf.at[b], sem).wait()
    pltpu.make_async_copy(m_hbm.at[pl.ds(0, BLK)], mbuf.at[b], sem).wait()


@functools.partial(
    pl.kernel,
    out_type=[
        jax.ShapeDtypeStruct((NW, C * KL), jnp.float32),
        jax.ShapeDtypeStruct((NW, KL), jnp.float32),
    ],
    mesh=_MESH,
    compiler_params=pltpu.CompilerParams(needs_layout_passes=False),
    scratch_types=[
        pltpu.VMEM((2, C, BLK), jnp.float32),
        pltpu.VMEM((2, BLK), jnp.int32),
        pltpu.VMEM((2, BLK), jnp.int32),
        pltpu.VMEM((C * KL,), jnp.float32),
        pltpu.VMEM((KL,), jnp.float32),
        pltpu.SemaphoreType.DMA,
        pltpu.SemaphoreType.DMA,
    ],
)
def _sc_pass1(e_hbm, t_hbm, m_hbm, sums_out, cnt_out,
              xbuf, tbuf, mbuf, acc, cacc, sem0, sem1):
    wid = _worker_id()
    base = wid * VPW
    lane = lax.broadcasted_iota(jnp.int32, (L,), 0)
    zero = jnp.zeros((L,), jnp.float32)
    ones = jnp.ones((L,), jnp.float32)
    stage = (e_hbm, t_hbm, m_hbm, xbuf, tbuf, mbuf)

    def zero_body(i, _):
        sl = pl.ds(i * L, L)
        cacc[sl] = zero
        for c in range(C):
            acc[pl.ds(c * KL + i * L, L)] = zero
        return 0

    lax.fori_loop(0, KL // L, zero_body, 0)

    _issue_chunk(*stage, 0, base, sem0)
    _issue_chunk(*stage, 1, base + BLK, sem1)

    def outer(g, _):
        for b, sem in ((0, sem0), (1, sem1)):
            j = g * 2 + b
            _drain_chunk(*stage, b, sem)

            def vec_body(i, _, b=b):
                sl = pl.ds(i * L, L)
                seg = jnp.where(mbuf[b, sl] > 0, tbuf[b, sl], 0)
                addr = seg * L + lane
                plsc.addupdate_scatter(cacc, [addr], ones)
                for c in range(C):
                    plsc.addupdate_scatter(acc, [addr + (c * KL)],
                                           xbuf[b, c, sl])
                return 0

            lax.fori_loop(0, BLK // L, vec_body, 0)
            nxt = base + jnp.minimum(j + 2, NCHUNK - 1) * BLK
            _issue_chunk(*stage, b, nxt, sem)
        return 0

    lax.fori_loop(0, NCHUNK // 2, outer, 0)
    _drain_chunk(*stage, 0, sem0)
    _drain_chunk(*stage, 1, sem1)
    pltpu.sync_copy(acc, sums_out.at[wid])
    pltpu.sync_copy(cacc, cnt_out.at[wid])


@functools.partial(
    pl.kernel,
    out_type=jax.ShapeDtypeStruct((NW, KL), jnp.float32),
    mesh=_MESH,
    compiler_params=pltpu.CompilerParams(needs_layout_passes=False),
    scratch_types=[
        pltpu.VMEM((2, C, BLK), jnp.float32),
        pltpu.VMEM((2, BLK), jnp.int32),
        pltpu.VMEM((2, BLK), jnp.int32),
        pltpu.VMEM((C * K,), jnp.float32),
        pltpu.VMEM((KL,), jnp.float32),
        pltpu.SemaphoreType.DMA,
        pltpu.SemaphoreType.DMA,
    ],
)
def _sc_pass2(e_hbm, t_hbm, m_hbm, means_hbm, term_out,
              xbuf, tbuf, mbuf, mtab, tacc, sem0, sem1):
    wid = _worker_id()
    base = wid * VPW
    lane = lax.broadcasted_iota(jnp.int32, (L,), 0)
    zero = jnp.zeros((L,), jnp.float32)
    stage = (e_hbm, t_hbm, m_hbm, xbuf, tbuf, mbuf)

    pltpu.sync_copy(means_hbm, mtab)

    def zero_body(i, _):
        tacc[pl.ds(i * L, L)] = zero
        return 0

    lax.fori_loop(0, KL // L, zero_body, 0)

    _issue_chunk(*stage, 0, base, sem0)
    _issue_chunk(*stage, 1, base + BLK, sem1)

    def outer(g, _):
        for b, sem in ((0, sem0), (1, sem1)):
            j = g * 2 + b
            _drain_chunk(*stage, b, sem)

            def vec_body(i, _, b=b):
                sl = pl.ds(i * L, L)
                seg = jnp.where(mbuf[b, sl] > 0, tbuf[b, sl], 0)
                d = zero
                for c in range(C):
                    mv = plsc.load_gather(mtab, [seg + (c * K)])
                    d = d + jnp.abs(xbuf[b, c, sl] - mv)
                plsc.addupdate_scatter(tacc, [seg * L + lane], d * d)
                return 0

            lax.fori_loop(0, BLK // L, vec_body, 0)
            nxt = base + jnp.minimum(j + 2, NCHUNK - 1) * BLK
            _issue_chunk(*stage, b, nxt, sem)
        return 0

    lax.fori_loop(0, NCHUNK // 2, outer, 0)
    _drain_chunk(*stage, 0, sem0)
    _drain_chunk(*stage, 1, sem1)
    pltpu.sync_copy(tacc, term_out.at[wid])


def _fold_matrices():
    jv = lax.broadcasted_iota(jnp.int32, (KL, K), 0)
    kv = lax.broadcasted_iota(jnp.int32, (KL, K), 1)
    F = ((jv // L) == kv).astype(jnp.float32)
    kv2 = lax.broadcasted_iota(jnp.int32, (K, KL), 0)
    jv2 = lax.broadcasted_iota(jnp.int32, (K, KL), 1)
    F2 = ((jv2 // L) == kv2).astype(jnp.float32)
    return F, F2


def _combine1_body(sums_ref, cnt_ref, means_ref, scal_ref):
    ones_w = jnp.ones((1, NW), jnp.float32)
    red = jnp.dot(ones_w, sums_ref[...], preferred_element_type=jnp.float32)
    cred = jnp.dot(ones_w, cnt_ref[...], preferred_element_type=jnp.float32)
    F, F2 = _fold_matrices()
    nt = (((1,), (1,)), ((), ()))
    cnt_row = jnp.dot(cred, F, preferred_element_type=jnp.float32)
    cnt_col = lax.dot_general(F2, cred, nt,
                              preferred_element_type=jnp.float32)
    safe_row = jnp.maximum(cnt_row, 1.0)
    safe_col = jnp.maximum(cnt_col, 1.0)
    obj_row = (cnt_row > 0.0) & (lax.broadcasted_iota(jnp.int32, (1, K), 1) > 0)
    obj_col = (cnt_col > 0.0) & (lax.broadcasted_iota(jnp.int32, (K, 1), 0) > 0)
    n_obj = jnp.sum(obj_row.astype(jnp.float32))
    dist = jnp.zeros((K, K), jnp.float32)
    nrm = jnp.zeros((K, 1), jnp.float32)
    for c in range(C):
        seg_c = red[:, c * KL:(c + 1) * KL]
        row_c = jnp.dot(seg_c, F, preferred_element_type=jnp.float32) / safe_row
        col_c = lax.dot_general(F2, seg_c, nt,
                                preferred_element_type=jnp.float32) / safe_col
        means_ref[c:c + 1, :] = row_c
        dist = dist + jnp.abs(col_c - row_c)
        nrm = nrm + jnp.abs(col_c)
    margin = jnp.maximum(2.0 * DELTA_D - dist, 0.0)
    ii = lax.broadcasted_iota(jnp.int32, (K, K), 0)
    jj = lax.broadcasted_iota(jnp.int32, (K, K), 1)
    pair = obj_col & obj_row & (ii != jj)
    loss_ext = jnp.sum(jnp.where(pair, margin * margin, 0.0))
    loss_ext = loss_ext / jnp.maximum(1.0, n_obj * (n_obj - 1.0))
    loss_nrm = jnp.sum(jnp.where(obj_col, nrm, 0.0)) / jnp.maximum(1.0, n_obj)
    scal_ref[...] = jnp.reshape(BETA * loss_ext + GAMMA * loss_nrm, (1, 1))


def _combine2_body(term_ref, cnt_ref, scal_ref, out_ref):
    ones_w = jnp.ones((1, NW), jnp.float32)
    tred = jnp.dot(ones_w, term_ref[...], preferred_element_type=jnp.float32)
    cred = jnp.dot(ones_w, cnt_ref[...], preferred_element_type=jnp.float32)
    F, _ = _fold_matrices()
    t_row = jnp.dot(tred, F, preferred_element_type=jnp.float32)
    cnt_row = jnp.dot(cred, F, preferred_element_type=jnp.float32)
    per_obj = t_row / jnp.maximum(cnt_row, 1.0)
    obj_row = (cnt_row > 0.0) & (lax.broadcasted_iota(jnp.int32, (1, K), 1) > 0)
    n_obj = jnp.sum(obj_row.astype(jnp.float32))
    loss_int = jnp.sum(jnp.where(obj_row, per_obj, 0.0))
    loss_int = loss_int / jnp.maximum(1.0, n_obj)
    out_ref[...] = jnp.reshape(ALPHA * loss_int, (1, 1)) + scal_ref[...]


def _combine1(sums_part, cnt_part):
    return pl.pallas_call(
        _combine1_body,
        out_shape=[
            jax.ShapeDtypeStruct((C, K), jnp.float32),
            jax.ShapeDtypeStruct((1, 1), jnp.float32),
        ],
    )(sums_part, cnt_part)


def _combine2(term_part, cnt_part, scal):
    return pl.pallas_call(
        _combine2_body,
        out_shape=jax.ShapeDtypeStruct((1, 1), jnp.float32),
    )(term_part, cnt_part, scal)


def kernel(embd, trgt, mask):
    e2 = embd.reshape(C, N)
    t1 = trgt.reshape(N).astype(jnp.int32)
    m1 = mask.reshape(N).astype(jnp.int32)
    sums_part, cnt_part = _sc_pass1(e2, t1, m1)
    means_ck, scal = _combine1(sums_part, cnt_part)
    term_part = _sc_pass2(e2, t1, m1, means_ck.reshape(C * K))
    out = _combine2(term_part, cnt_part, scal)
    return out[0, 0]


# speedup vs baseline: 28.6132x; 5.5585x over previous
import functools

import jax
import jax.numpy as jnp
from jax import lax
from jax.experimental import pallas as pl
from jax.experimental.pallas import tpu as pltpu
from jax.experimental.pallas import tpu_sc as plsc

K = 64
C = 16
N = 32 * 256 * 256
ALPHA = 1.0
BETA = 1.0
GAMMA = 0.001
DELTA_D = 1.5

NC = 2
NS = 16
L = 16
NW = NC * NS
KL = K * L
VPW = N // NW
BLK = 2048
NCHUNK = VPW // BLK

_MESH = plsc.VectorSubcoreMesh(
    core_axis_name="c", subcore_axis_name="s", num_cores=NC, num_subcores=NS
)


def _worker_id():
    return lax.axis_index("s") * NC + lax.axis_index("c")


def _issue_chunk(e_hbm, t_hbm, m_hbm, xbuf, tbuf, mbuf, b, off, sem):
    for c in range(C):
        pltpu.async_copy(e_hbm.at[pl.ds(c * N + off, BLK)], xbuf.at[b, c], sem)
    pltpu.async_copy(t_hbm.at[pl.ds(off, BLK)], tbuf.at[b], sem)
    pltpu.async_copy(m_hbm.at[pl.ds(off, BLK)], mbuf.at[b], sem)


def _drain_chunk(e_hbm, t_hbm, m_hbm, xbuf, tbuf, mbuf, b, sem):
    for c in range(C):
        pltpu.make_async_copy(
            e_hbm.at[pl.ds(0, BLK)], xbuf.at[b, c], sem).wait()
    pltpu.make_async_copy(t_hbm.at[pl.ds(0, BLK)], tbuf.at[b], sem).wait()
    pltpu.make_async_copy(m_hbm.at[pl.ds(0, BLK)], mbuf.at[b], sem).wait()


@functools.partial(
    pl.kernel,
    out_type=[
        jax.ShapeDtypeStruct((NW, C * KL), jnp.float32),
        jax.ShapeDtypeStruct((NW, KL), jnp.float32),
    ],
    mesh=_MESH,
    compiler_params=pltpu.CompilerParams(needs_layout_passes=False),
    scratch_types=[
        pltpu.VMEM((2, C, BLK), jnp.float32),
        pltpu.VMEM((2, BLK), jnp.int32),
        pltpu.VMEM((2, BLK), jnp.int32),
        pltpu.VMEM((C * KL,), jnp.float32),
        pltpu.VMEM((KL,), jnp.float32),
        pltpu.SemaphoreType.DMA,
        pltpu.SemaphoreType.DMA,
    ],
)
def _sc_pass1(e_hbm, t_hbm, m_hbm, sums_out, cnt_out,
              xbuf, tbuf, mbuf, acc, cacc, sem0, sem1):
    wid = _worker_id()
    base = wid * VPW
    lane = lax.broadcasted_iota(jnp.int32, (L,), 0)
    zero = jnp.zeros((L,), jnp.float32)
    ones = jnp.ones((L,), jnp.float32)
    stage = (e_hbm, t_hbm, m_hbm, xbuf, tbuf, mbuf)

    def zero_body(i, _):
        sl = pl.ds(i * L, L)
        cacc[sl] = zero
        for c in range(C):
            acc[pl.ds(c * KL + i * L, L)] = zero
        return 0

    lax.fori_loop(0, KL // L, zero_body, 0)

    _issue_chunk(*stage, 0, base, sem0)
    _issue_chunk(*stage, 1, base + BLK, sem1)

    def outer(g, _):
        for b, sem in ((0, sem0), (1, sem1)):
            j = g * 2 + b
            _drain_chunk(*stage, b, sem)

            def vec_body(i, _, b=b):
                for u in range(4):
                    sl = pl.ds((i * 4 + u) * L, L)
                    seg = jnp.where(mbuf[b, sl] > 0, tbuf[b, sl], 0)
                    addr = seg * L + lane
                    plsc.addupdate_scatter(cacc, [addr], ones)
                    for c in range(C):
                        plsc.addupdate_scatter(acc, [addr + (c * KL)],
                                               xbuf[b, c, sl])
                return 0

            lax.fori_loop(0, BLK // L // 4, vec_body, 0)
            nxt = base + jnp.minimum(j + 2, NCHUNK - 1) * BLK
            _issue_chunk(*stage, b, nxt, sem)
        return 0

    lax.fori_loop(0, NCHUNK // 2, outer, 0)
    _drain_chunk(*stage, 0, sem0)
    _drain_chunk(*stage, 1, sem1)
    pltpu.sync_copy(acc, sums_out.at[wid])
    pltpu.sync_copy(cacc, cnt_out.at[wid])


@functools.partial(
    pl.kernel,
    out_type=jax.ShapeDtypeStruct((NW, KL), jnp.float32),
    mesh=_MESH,
    compiler_params=pltpu.CompilerParams(needs_layout_passes=False),
    scratch_types=[
        pltpu.VMEM((2, C, BLK), jnp.float32),
        pltpu.VMEM((2, BLK), jnp.int32),
        pltpu.VMEM((2, BLK), jnp.int32),
        pltpu.VMEM((C * KL,), jnp.float32),
        pltpu.VMEM((KL,), jnp.float32),
        pltpu.SemaphoreType.DMA,
        pltpu.SemaphoreType.DMA,
    ],
)
def _sc_pass2(e_hbm, t_hbm, m_hbm, means_hbm, term_out,
              xbuf, tbuf, mbuf, mtab, tacc, sem0, sem1):
    wid = _worker_id()
    base = wid * VPW
    lane = lax.broadcasted_iota(jnp.int32, (L,), 0)
    zero = jnp.zeros((L,), jnp.float32)
    stage = (e_hbm, t_hbm, m_hbm, xbuf, tbuf, mbuf)

    pltpu.sync_copy(means_hbm, mtab)

    def zero_body(i, _):
        tacc[pl.ds(i * L, L)] = zero
        return 0

    lax.fori_loop(0, KL // L, zero_body, 0)

    _issue_chunk(*stage, 0, base, sem0)
    _issue_chunk(*stage, 1, base + BLK, sem1)

    def outer(g, _):
        for b, sem in ((0, sem0), (1, sem1)):
            j = g * 2 + b
            _drain_chunk(*stage, b, sem)

            def vec_body(i, _, b=b):
                for u in range(2):
                    sl = pl.ds((i * 2 + u) * L, L)
                    seg = jnp.where(mbuf[b, sl] > 0, tbuf[b, sl], 0)
                    addr = seg * L + lane
                    parts = [jnp.abs(xbuf[b, c, sl]
                                     - plsc.load_gather(mtab, [addr + (c * KL)]))
                             for c in range(C)]
                    while len(parts) > 1:
                        parts = [parts[k] + parts[k + 1]
                                 for k in range(0, len(parts), 2)]
                    d = parts[0]
                    plsc.addupdate_scatter(tacc, [addr], d * d)
                return 0

            lax.fori_loop(0, BLK // L // 2, vec_body, 0)
            nxt = base + jnp.minimum(j + 2, NCHUNK - 1) * BLK
            _issue_chunk(*stage, b, nxt, sem)
        return 0

    lax.fori_loop(0, NCHUNK // 2, outer, 0)
    _drain_chunk(*stage, 0, sem0)
    _drain_chunk(*stage, 1, sem1)
    pltpu.sync_copy(tacc, term_out.at[wid])


def _fold_matrices():
    jv = lax.broadcasted_iota(jnp.int32, (KL, K), 0)
    kv = lax.broadcasted_iota(jnp.int32, (KL, K), 1)
    F = ((jv // L) == kv).astype(jnp.float32)
    kv2 = lax.broadcasted_iota(jnp.int32, (K, KL), 0)
    jv2 = lax.broadcasted_iota(jnp.int32, (K, KL), 1)
    F2 = ((jv2 // L) == kv2).astype(jnp.float32)
    return F, F2


def _combine1_body(sums_ref, cnt_ref, means_ref, scal_ref):
    ones_w = jnp.ones((1, NW), jnp.float32)
    red = jnp.dot(ones_w, sums_ref[...], preferred_element_type=jnp.float32)
    cred = jnp.dot(ones_w, cnt_ref[...], preferred_element_type=jnp.float32)
    F, F2 = _fold_matrices()
    nt = (((1,), (1,)), ((), ()))
    cnt_row = jnp.dot(cred, F, preferred_element_type=jnp.float32)
    cnt_col = lax.dot_general(F2, cred, nt,
                              preferred_element_type=jnp.float32)
    safe_row = jnp.maximum(cnt_row, 1.0)
    safe_col = jnp.maximum(cnt_col, 1.0)
    obj_row = (cnt_row > 0.0) & (lax.broadcasted_iota(jnp.int32, (1, K), 1) > 0)
    obj_col = (cnt_col > 0.0) & (lax.broadcasted_iota(jnp.int32, (K, 1), 0) > 0)
    n_obj = jnp.sum(obj_row.astype(jnp.float32))
    dist = jnp.zeros((K, K), jnp.float32)
    nrm = jnp.zeros((K, 1), jnp.float32)
    for c in range(C):
        seg_c = red[:, c * KL:(c + 1) * KL]
        row_c = jnp.dot(seg_c, F, preferred_element_type=jnp.float32) / safe_row
        col_c = lax.dot_general(F2, seg_c, nt,
                                preferred_element_type=jnp.float32) / safe_col
        means_ref[pl.ds(c * K, K), :] = jnp.broadcast_to(col_c, (K, L))
        dist = dist + jnp.abs(col_c - row_c)
        nrm = nrm + jnp.abs(col_c)
    margin = jnp.maximum(2.0 * DELTA_D - dist, 0.0)
    ii = lax.broadcasted_iota(jnp.int32, (K, K), 0)
    jj = lax.broadcasted_iota(jnp.int32, (K, K), 1)
    pair = obj_col & obj_row & (ii != jj)
    loss_ext = jnp.sum(jnp.where(pair, margin * margin, 0.0))
    loss_ext = loss_ext / jnp.maximum(1.0, n_obj * (n_obj - 1.0))
    loss_nrm = jnp.sum(jnp.where(obj_col, nrm, 0.0)) / jnp.maximum(1.0, n_obj)
    scal_ref[...] = jnp.reshape(BETA * loss_ext + GAMMA * loss_nrm, (1, 1))


def _combine2_body(term_ref, cnt_ref, scal_ref, out_ref):
    ones_w = jnp.ones((1, NW), jnp.float32)
    tred = jnp.dot(ones_w, term_ref[...], preferred_element_type=jnp.float32)
    cred = jnp.dot(ones_w, cnt_ref[...], preferred_element_type=jnp.float32)
    F, _ = _fold_matrices()
    t_row = jnp.dot(tred, F, preferred_element_type=jnp.float32)
    cnt_row = jnp.dot(cred, F, preferred_element_type=jnp.float32)
    per_obj = t_row / jnp.maximum(cnt_row, 1.0)
    obj_row = (cnt_row > 0.0) & (lax.broadcasted_iota(jnp.int32, (1, K), 1) > 0)
    n_obj = jnp.sum(obj_row.astype(jnp.float32))
    loss_int = jnp.sum(jnp.where(obj_row, per_obj, 0.0))
    loss_int = loss_int / jnp.maximum(1.0, n_obj)
    out_ref[...] = jnp.reshape(ALPHA * loss_int, (1, 1)) + scal_ref[...]


def _combine1(sums_part, cnt_part):
    return pl.pallas_call(
        _combine1_body,
        out_shape=[
            jax.ShapeDtypeStruct((C * K, L), jnp.float32),
            jax.ShapeDtypeStruct((1, 1), jnp.float32),
        ],
    )(sums_part, cnt_part)


def _combine2(term_part, cnt_part, scal):
    return pl.pallas_call(
        _combine2_body,
        out_shape=jax.ShapeDtypeStruct((1, 1), jnp.float32),
    )(term_part, cnt_part, scal)


def kernel(embd, trgt, mask):
    e2 = embd.reshape(C * N)
    t1 = trgt.reshape(N).astype(jnp.int32)
    m1 = mask.reshape(N).astype(jnp.int32)
    sums_part, cnt_part = _sc_pass1(e2, t1, m1)
    means_rep, scal = _combine1(sums_part, cnt_part)
    term_part = _sc_pass2(e2, t1, m1, means_rep.reshape(C * KL))
    out = _combine2(term_part, cnt_part, scal)
    return out[0, 0]
